# Initial kernel scaffold; baseline (speedup 1.0000x reference)
import jax
import jax.numpy as jnp
from jax.experimental import pallas as pl


def kernel(q, v, embeddings):
    raise NotImplementedError("write your pallas kernel here")


# speedup vs baseline: 8.1828x; 8.1828x over previous
import functools

import jax
import jax.numpy as jnp
from jax import lax
from jax.experimental import pallas as pl
from jax.experimental.pallas import tpu as pltpu
from jax.experimental.pallas import tpu_sc as plsc

NC = 2
NS = 16
NW = NC * NS

Q_LEN = 2048
V_LEN = 2048
D = 32
VOCAB = 257
MAX_POS = (VOCAB - 1) // 2

M_ROWS = 4224
MID = Q_LEN - MAX_POS
ROWS_PER_W = Q_LEN // NW
WIN_ROWS = 2176


def _band_build(embeddings):
  def body(emb_ref, out_ref):
    row0 = emb_ref[0:1, :]
    out_ref[0:MID, :] = jnp.broadcast_to(row0, (MID, D))
    out_ref[MID:MID + VOCAB - 1, :] = emb_ref[0:VOCAB - 1, :]
    row_last = emb_ref[VOCAB - 1:VOCAB, :]
    out_ref[MID + VOCAB - 1:M_ROWS, :] = jnp.broadcast_to(
        row_last, (M_ROWS - (MID + VOCAB - 1), D))

  return pl.pallas_call(
      body,
      out_shape=jax.ShapeDtypeStruct((M_ROWS, D), jnp.float32),
  )(embeddings)


def _band_expand(m_hbm):
  mesh = plsc.VectorSubcoreMesh(core_axis_name="c", subcore_axis_name="s",
                                num_cores=NC)

  @functools.partial(
      pl.kernel,
      mesh=mesh,
      compiler_params=pltpu.CompilerParams(use_tc_tiling_on_sc=False),
      out_type=jax.ShapeDtypeStruct((Q_LEN, V_LEN, D), jnp.float32),
      scratch_types=[
          pltpu.VMEM((WIN_ROWS, D), jnp.float32),
          pltpu.SemaphoreType.DMA,
      ],
  )
  def k(m_ref, out_hbm, m_v, sem):
    wid = lax.axis_index("s") * NC + lax.axis_index("c")
    row0 = wid * ROWS_PER_W
    base = Q_LEN - (row0 + ROWS_PER_W)

    pltpu.async_copy(m_ref.at[pl.ds(base, WIN_ROWS)], m_v, sem).wait()

    def row_copy(r, carry):
      start = ROWS_PER_W - r
      pltpu.async_copy(
          m_v.at[pl.ds(start, V_LEN)],
          out_hbm.at[row0 + r],
          sem,
      ).wait()
      return carry

    lax.fori_loop(0, ROWS_PER_W, row_copy, 0)

  return k(m_hbm)


def kernel(q, v, embeddings):
  del q, v
  return _band_expand(_band_build(embeddings))
